# Initial kernel scaffold; baseline (speedup 1.0000x reference)
import jax
import jax.numpy as jnp
from jax.experimental import pallas as pl


def kernel(x, edge_index, target_indices, proj_W, proj_b, gcn_W1, gcn_b1, gcn_W2, gcn_b2, gru_Wih, gru_Whh, gru_bih, gru_bhh, att_W, att_b, pred_W1, pred_b1, pred_W2, pred_b2):
    raise NotImplementedError("write your pallas kernel here")


# speedup vs baseline: 4.0689x; 4.0689x over previous
import functools

import jax
import jax.numpy as jnp
from jax import lax
from jax.experimental import pallas as pl
from jax.experimental.pallas import tpu as pltpu
from jax.experimental.pallas import tpu_sc as plsc

N = 10000
E = 320000
T = 6
B = 1024
D = 128
NC, NS = 2, 16
NW = NC * NS
CHUNK = 128
CH = 80
PCH = 40
EP = NW * CH * CHUNK
ACC_N = N + 16
RSTRIDE = 624
RZ = 656
RO = 640
BLK = 1000
TGT_PT = B // NW

_mesh = plsc.VectorSubcoreMesh(core_axis_name="c", subcore_axis_name="s")


@functools.partial(
    pl.kernel,
    out_type=jax.ShapeDtypeStruct((NC, N, 16), jnp.float32),
    mesh=_mesh,
    scratch_types=[
        pltpu.VMEM((CH, CHUNK), jnp.int32),
        pltpu.VMEM((CHUNK, 16), jnp.float32),
        pltpu.VMEM_SHARED((ACC_N, 16), jnp.float32),
    ],
)
def _deg_kernel(dst_hbm, out_hbm, idx_v, ones_v, acc):
    cid = lax.axis_index("c")
    sid = lax.axis_index("s")
    wid = sid * NC + cid

    def _fill(i, val):
        ones_v[i] = jnp.full((16,), val, jnp.float32)
        return val

    lax.fori_loop(0, CHUNK, _fill, 0.0)
    zb = sid * RSTRIDE
    for k in range(RZ // CHUNK):
        pltpu.sync_copy(ones_v, acc.at[pl.ds(zb + k * CHUNK, CHUNK)])
    pltpu.sync_copy(ones_v.at[pl.ds(0, RZ % CHUNK)],
                    acc.at[pl.ds(zb + (RZ // CHUNK) * CHUNK, RZ % CHUNK)])
    plsc.subcore_barrier()

    lax.fori_loop(0, CHUNK, _fill, 1.0)
    pltpu.sync_copy(dst_hbm.at[wid], idx_v)

    def _body(j, carry):
        pltpu.sync_copy(ones_v, acc.at[idx_v.at[j]], add=True)
        return carry

    lax.fori_loop(0, CH, _body, 0)
    plsc.subcore_barrier()
    ob = sid * RSTRIDE
    pltpu.sync_copy(acc.at[pl.ds(ob, RO)], out_hbm.at[cid, pl.ds(ob, RO)])


@functools.partial(
    pl.kernel,
    out_type=jax.ShapeDtypeStruct((NC, N, D), jnp.float32),
    mesh=_mesh,
    scratch_types=[
        pltpu.VMEM((PCH, CHUNK), jnp.int32),
        pltpu.VMEM((PCH, CHUNK), jnp.int32),
        pltpu.VMEM((CHUNK, D), jnp.float32),
        pltpu.VMEM((CHUNK, D), jnp.float32),
        pltpu.VMEM_SHARED((ACC_N, D), jnp.float32),
        pltpu.SemaphoreType.DMA,
        pltpu.SemaphoreType.DMA,
    ],
)
def _prop_kernel(g_hbm, src_hbm, dst_hbm, out_hbm,
                 sidx, didx, bufa, bufb, acc, sema, semb):
    cid = lax.axis_index("c")
    sid = lax.axis_index("s")
    wid = sid * NC + cid

    def _zrow(i, carry):
        for k in range(D // 16):
            bufa[i, pl.ds(k * 16, 16)] = jnp.zeros((16,), jnp.float32)
        return carry

    lax.fori_loop(0, CHUNK, _zrow, 0)
    zb = sid * RSTRIDE
    for k in range(RZ // CHUNK):
        pltpu.sync_copy(bufa, acc.at[pl.ds(zb + k * CHUNK, CHUNK)])
    pltpu.sync_copy(bufa.at[pl.ds(0, RZ % CHUNK)],
                    acc.at[pl.ds(zb + (RZ // CHUNK) * CHUNK, RZ % CHUNK)])
    plsc.subcore_barrier()

    for p in range(CH // PCH):
        pltpu.sync_copy(src_hbm.at[wid, pl.ds(p * PCH, PCH)], sidx)
        pltpu.sync_copy(dst_hbm.at[wid, pl.ds(p * PCH, PCH)], didx)
        pltpu.async_copy(g_hbm.at[sidx.at[0]], bufa, sema)

        def _body(j, carry):
            ca = 2 * j
            pltpu.make_async_copy(g_hbm.at[sidx.at[ca]], bufa, sema).wait()
            pltpu.async_copy(g_hbm.at[sidx.at[ca + 1]], bufb, semb)
            pltpu.sync_copy(bufa, acc.at[didx.at[ca]], add=True)
            pltpu.make_async_copy(g_hbm.at[sidx.at[ca + 1]], bufb, semb).wait()

            @pl.when(ca + 2 < PCH)
            def _():
                pltpu.async_copy(g_hbm.at[sidx.at[ca + 2]], bufa, sema)

            pltpu.sync_copy(bufb, acc.at[didx.at[ca + 1]], add=True)
            return carry

        lax.fori_loop(0, PCH // 2, _body, 0)
    plsc.subcore_barrier()
    ob = sid * RSTRIDE
    pltpu.sync_copy(acc.at[pl.ds(ob, RO)], out_hbm.at[cid, pl.ds(ob, RO)])


def _make_gather_kernel():
    @functools.partial(
        pl.kernel,
        out_type=jax.ShapeDtypeStruct((T, 2, B, D), jnp.float32),
        mesh=_mesh,
        scratch_types=[
            pltpu.VMEM((TGT_PT,), jnp.int32),
            pltpu.VMEM((TGT_PT, D), jnp.float32),
            pltpu.SemaphoreType.DMA,
        ],
    )
    def _gather_kernel(tgt_hbm, *args):
        hs = args[:2 * T]
        out_hbm = args[2 * T]
        idx_v, buf, sem = args[2 * T + 1:]
        cid = lax.axis_index("c")
        sid = lax.axis_index("s")
        wid = sid * NC + cid
        base = wid * TGT_PT
        pltpu.sync_copy(tgt_hbm.at[pl.ds(base, TGT_PT)], idx_v)
        for t in range(T):
            for l in range(2):
                pltpu.async_copy(hs[t * 2 + l].at[idx_v], buf, sem).wait()
                pltpu.sync_copy(buf, out_hbm.at[t, l, pl.ds(base, TGT_PT)])

    return _gather_kernel


_gather_kernel = _make_gather_kernel()


def _dinv(degp):
    return lax.rsqrt(degp[0, :, 0:1] + degp[1, :, 0:1] + 1.0)


def _stage1_body(x_ref, pw_ref, pb_ref, w1_ref, degp_ref, g1_ref):
    px = jnp.maximum(
        jnp.dot(x_ref[...], pw_ref[...], preferred_element_type=jnp.float32)
        + pb_ref[...], 0.0)
    g1_ref[...] = jnp.dot(px, w1_ref[...],
                          preferred_element_type=jnp.float32) * _dinv(degp_ref[...])


def _stage2_body(s_ref, g_ref, degp_ref, b1_ref, w2_ref, h1_ref, g2_ref):
    dinv = _dinv(degp_ref[...])
    h1 = jnp.maximum(
        (s_ref[0] + s_ref[1] + g_ref[...]) * dinv + b1_ref[...], 0.0)
    h1_ref[...] = h1
    g2_ref[...] = jnp.dot(h1, w2_ref[...],
                          preferred_element_type=jnp.float32) * dinv


def _stage3_body(s_ref, g_ref, degp_ref, b2_ref, h2_ref):
    dinv = _dinv(degp_ref[...])
    h2_ref[...] = jnp.maximum(
        (s_ref[0] + s_ref[1] + g_ref[...]) * dinv + b2_ref[...], 0.0)


_row = lambda: pl.BlockSpec((BLK, D), lambda i: (i, 0))
_whole = lambda shape: pl.BlockSpec(shape, lambda i: tuple(0 for _ in shape))
_degs = pl.BlockSpec((NC, BLK, 16), lambda i: (0, i, 0))
_sspec = pl.BlockSpec((NC, BLK, D), lambda i: (0, i, 0))
_GRID = N // BLK


def _stage1(x_t, pwT, pb, w1T, degp):
    return pl.pallas_call(
        _stage1_body,
        grid=(_GRID,),
        in_specs=[_row(), _whole((D, D)), _whole((1, D)), _whole((D, D)), _degs],
        out_specs=_row(),
        out_shape=jax.ShapeDtypeStruct((N, D), jnp.float32),
    )(x_t, pwT, pb, w1T, degp)


def _stage2(s, g1, degp, b1, w2T):
    return pl.pallas_call(
        _stage2_body,
        grid=(_GRID,),
        in_specs=[_sspec, _row(), _degs, _whole((1, D)), _whole((D, D))],
        out_specs=(_row(), _row()),
        out_shape=(jax.ShapeDtypeStruct((N, D), jnp.float32),
                   jax.ShapeDtypeStruct((N, D), jnp.float32)),
    )(s, g1, degp, b1, w2T)


def _stage3(s, g2, degp, b2):
    return pl.pallas_call(
        _stage3_body,
        grid=(_GRID,),
        in_specs=[_sspec, _row(), _degs, _whole((1, D))],
        out_specs=_row(),
        out_shape=jax.ShapeDtypeStruct((N, D), jnp.float32),
    )(s, g2, degp, b2)


GB = 256


def _gru_body(tgt_ref, wih1_ref, wih2_ref, whh_ref, bih_ref, bhh_ref,
              attw_ref, attb_ref, pw1_ref, pb1_ref, pw2_ref, pb2_ref,
              pred_ref, wts_ref):
    h = jnp.zeros((GB, D), jnp.float32)
    hs = []
    for t in range(T):
        gi = (jnp.dot(tgt_ref[t, 0], wih1_ref[...],
                      preferred_element_type=jnp.float32)
              + jnp.dot(tgt_ref[t, 1], wih2_ref[...],
                        preferred_element_type=jnp.float32) + bih_ref[...])
        gh = jnp.dot(h, whh_ref[...],
                     preferred_element_type=jnp.float32) + bhh_ref[...]
        r = jax.nn.sigmoid(gi[:, 0:D] + gh[:, 0:D])
        z = jax.nn.sigmoid(gi[:, D:2 * D] + gh[:, D:2 * D])
        n = jnp.tanh(gi[:, 2 * D:3 * D] + r * gh[:, 2 * D:3 * D])
        h = (1.0 - z) * n + z * h
        hs.append(h)
    scores = [jnp.tanh(jnp.sum(ht * attw_ref[...], axis=1, keepdims=True)
                       + attb_ref[...]) for ht in hs]
    m = scores[0]
    for s in scores[1:]:
        m = jnp.maximum(m, s)
    es = [jnp.exp(s - m) for s in scores]
    zsum = es[0]
    for e in es[1:]:
        zsum = zsum + e
    ws = [e / zsum for e in es]
    final = ws[0] * hs[0]
    for w, ht in zip(ws[1:], hs[1:]):
        final = final + w * ht
    hidden = jnp.maximum(
        jnp.dot(final, pw1_ref[...], preferred_element_type=jnp.float32)
        + pb1_ref[...], 0.0)
    pred_ref[...] = jnp.sum(hidden * pw2_ref[...], axis=1, keepdims=True) \
        + pb2_ref[...]
    wts_ref[...] = jnp.concatenate(ws, axis=1)


def _gru_head(tgt, wih1T, wih2T, whhT, bih, bhh, attw, attb, pw1T, pb1, pw2, pb2):
    return pl.pallas_call(
        _gru_body,
        grid=(B // GB,),
        in_specs=[
            pl.BlockSpec((T, 2, GB, D), lambda i: (0, 0, i, 0)),
            _whole((D, 3 * D)), _whole((D, 3 * D)), _whole((D, 3 * D)),
            _whole((1, 3 * D)), _whole((1, 3 * D)),
            _whole((1, D)), _whole((1, 1)),
            _whole((D, 16)), _whole((1, 16)), _whole((1, 16)), _whole((1, 1)),
        ],
        out_specs=(pl.BlockSpec((GB, 1), lambda i: (i, 0)),
                   pl.BlockSpec((GB, T), lambda i: (i, 0))),
        out_shape=(jax.ShapeDtypeStruct((B, 1), jnp.float32),
                   jax.ShapeDtypeStruct((B, T), jnp.float32)),
    )(tgt, wih1T, wih2T, whhT, bih, bhh, attw, attb, pw1T, pb1, pw2, pb2)


def kernel(x, edge_index, target_indices, proj_W, proj_b, gcn_W1, gcn_b1,
           gcn_W2, gcn_b2, gru_Wih, gru_Whh, gru_bih, gru_bhh,
           att_W, att_b, pred_W1, pred_b1, pred_W2, pred_b2):
    assert x.shape == (T, N, D) and edge_index.shape == (2, E)

    src = edge_index[0].astype(jnp.int32)
    dst = edge_index[1].astype(jnp.int32)
    pad = EP - E
    src_p = jnp.concatenate([src, jnp.zeros((pad,), jnp.int32)]).reshape(NW, CH, CHUNK)
    dst_p = jnp.concatenate([dst, jnp.full((pad,), N, jnp.int32)]).reshape(NW, CH, CHUNK)
    tgt_idx = target_indices.astype(jnp.int32)

    pwT = proj_W.T
    w1T = gcn_W1.T
    w2T = gcn_W2.T
    pb = proj_b.reshape(1, D)
    b1 = gcn_b1.reshape(1, D)
    b2 = gcn_b2.reshape(1, D)
    wihT = gru_Wih.T
    wih1T = wihT[:D]
    wih2T = wihT[D:]
    whhT = gru_Whh.T
    bih = gru_bih.reshape(1, 3 * D)
    bhh = gru_bhh.reshape(1, 3 * D)
    attw = att_W.reshape(1, D)
    attb = att_b.reshape(1, 1)
    pw1T = pred_W1.T
    pb1 = pred_b1.reshape(1, 16)
    pw2 = pred_W2.reshape(1, 16)
    pb2 = pred_b2.reshape(1, 1)

    degp = _deg_kernel(dst_p)

    hts = []
    for t in range(T):
        g1 = _stage1(x[t], pwT, pb, w1T, degp)
        s1 = _prop_kernel(g1, src_p, dst_p)
        h1, g2 = _stage2(s1, g1, degp, b1, w2T)
        s2 = _prop_kernel(g2, src_p, dst_p)
        h2 = _stage3(s2, g2, degp, b2)
        hts.extend([h1, h2])

    tgt = _gather_kernel(tgt_idx, *hts)
    pred, wts = _gru_head(tgt, wih1T, wih2T, whhT, bih, bhh,
                          attw, attb, pw1T, pb1, pw2, pb2)
    return (pred, wts)
